# baseline (speedup 1.0000x reference)
import jax
import jax.numpy as jnp
from jax.experimental import pallas as pl
from jax.experimental.pallas import tpu as pltpu

N = 64
E = 128
H = 128
NUM_MSG = 6
NUM_ATOM_TYPES = 171
NUM_BOND_TYPES = 200
OUT_DIM = 2
NB = E // 2
BM = 8

_BF = jnp.bfloat16
_DN = (((1,), (0,)), ((), ()))


def _dot(a, b):
    return jax.lax.dot_general(a.astype(_BF), b, _DN,
                               preferred_element_type=jnp.float32)


def _mpnn_body(atom3, bond3, src3, dst3, bidx,
               aemb_r, bemb_r, memb_r, bdew_r,
               ew1_r, eb1_r, ew2_r, eb2_r,
               mw1_r, mb1_r, mw2_r, mb2_r,
               uw1_r, ub1_r, uw2_r, ub2_r,
               out):
    f32 = jnp.float32
    relu = lambda x: jnp.maximum(x, 0.0)

    aemb = aemb_r[...]
    bemb = bemb_r[...]
    memb = memb_r[...]
    bdew = bdew_r[...]

    A3 = atom3[...]
    B3 = bond3[...]
    S3 = src3[...]
    D3 = dst3[...]
    BI = bidx[...]

    iota_an = jax.lax.broadcasted_iota(jnp.int32, (N, NUM_ATOM_TYPES), 1)
    iota_bn = jax.lax.broadcasted_iota(jnp.int32, (E, NUM_BOND_TYPES), 1)
    iota_en = jax.lax.broadcasted_iota(jnp.int32, (E, N), 1)
    iota_ne = jax.lax.broadcasted_iota(jnp.int32, (N, E), 0)
    iota_be = jax.lax.broadcasted_iota(jnp.int32, (NB, E), 0)

    atom_parts, bond_parts = [], []
    amask_parts, bmask_parts = [], []
    src_g, dst_g, src_s, agg_oh, mean_lk = [], [], [], [], []
    for m in range(BM):
        a = A3[m]
        b = B3[m]
        s = S3[m]
        d = D3[m]
        bi = BI[m:m + 1, :]
        a_oh = (iota_an == a).astype(_BF)
        b_oh = (iota_bn == b).astype(_BF)
        atom_parts.append(_dot(a_oh, aemb))
        bond_parts.append(_dot(b_oh, bemb))
        mean_lk.append(_dot(b_oh, memb))
        amask_parts.append((a != 0).astype(f32))
        bmask_parts.append((b != 0).astype(f32))
        src_g.append((iota_en == s).astype(_BF))
        dst_g.append((iota_en == d).astype(_BF))
        src_s.append((iota_ne == s.T).astype(_BF))
        agg_oh.append((iota_be == bi).astype(_BF))

    atom_state = jnp.concatenate(atom_parts, axis=0)
    bond_state = jnp.concatenate(bond_parts, axis=0)
    amask = jnp.concatenate(amask_parts, axis=0)
    bmask = jnp.concatenate(bmask_parts, axis=0)

    for i in range(NUM_MSG):
        W1 = ew1_r[i]
        src_atom = jnp.concatenate(
            [_dot(src_g[m], atom_state[m * N:(m + 1) * N].astype(_BF))
             for m in range(BM)], axis=0)
        dst_atom = jnp.concatenate(
            [_dot(dst_g[m], atom_state[m * N:(m + 1) * N].astype(_BF))
             for m in range(BM)], axis=0)
        h = relu(_dot(bond_state, W1[0:H])
                 + _dot(src_atom, W1[H:2 * H])
                 + _dot(dst_atom, W1[2 * H:3 * H])
                 + eb1_r[i])
        nb = _dot(h, ew2_r[i]) + eb2_r[i]
        bond_state = bond_state + nb * bmask
        if i < NUM_MSG - 1:
            M1 = mw1_r[i]
            h2 = relu(_dot(dst_atom, M1[0:H]) + _dot(bond_state, M1[H:2 * H])
                      + mb1_r[i])
            msg = (_dot(h2, mw2_r[i]) + mb2_r[i]) * bmask
            agg = jnp.concatenate(
                [_dot(src_s[m], msg[m * E:(m + 1) * E].astype(_BF))
                 for m in range(BM)], axis=0)
            na = relu(_dot(agg, uw1_r[i]) + ub1_r[i])
            na = _dot(na, uw2_r[i]) + ub2_r[i]
            atom_state = atom_state + na * amask

    masked = bond_state * bmask
    for m in range(BM):
        msl = masked[m * E:(m + 1) * E].astype(_BF)
        feat = _dot(agg_oh[m], msl)
        cnt = jnp.maximum(_dot(agg_oh[m], bmask_parts[m].astype(_BF)), 1.0)
        magg = _dot(agg_oh[m], (mean_lk[m] * bmask_parts[m]))
        out[m] = _dot(feat / cnt, bdew) + magg / cnt


@jax.jit
def kernel(atom, bond, connectivity, bond_indices, params):
    B = atom.shape[0]
    atom = atom.astype(jnp.int32)
    bond = bond.astype(jnp.int32)
    connectivity = connectivity.astype(jnp.int32)
    bond_indices = bond_indices.astype(jnp.int32)

    atom3 = atom.reshape(B, N, 1)
    bond3 = bond.reshape(B, E, 1)
    src3 = connectivity[:, :, 0].reshape(B, E, 1)
    dst3 = connectivity[:, :, 1].reshape(B, E, 1)

    bf = lambda x: x.astype(_BF)
    ew1 = bf(jnp.stack([p['w1'].T for p in params['edge']]))
    eb1 = jnp.stack([p['b1'].reshape(1, -1) for p in params['edge']])
    ew2 = bf(jnp.stack([p['w2'].T for p in params['edge']]))
    eb2 = jnp.stack([p['b2'].reshape(1, -1) for p in params['edge']])
    mw1 = bf(jnp.stack([p['mw1'].T for p in params['node']]))
    mb1 = jnp.stack([p['mb1'].reshape(1, -1) for p in params['node']])
    mw2 = bf(jnp.stack([p['mw2'].T for p in params['node']]))
    mb2 = jnp.stack([p['mb2'].reshape(1, -1) for p in params['node']])
    uw1 = bf(jnp.stack([p['uw1'].T for p in params['node']]))
    ub1 = jnp.stack([p['ub1'].reshape(1, -1) for p in params['node']])
    uw2 = bf(jnp.stack([p['uw2'].T for p in params['node']]))
    ub2 = jnp.stack([p['ub2'].reshape(1, -1) for p in params['node']])
    bdew = bf(params['bde_no_mean_w'].T)
    aemb = bf(params['atom_emb'])
    bemb = bf(params['bond_emb'])
    memb = bf(params['bde_mean_emb'])

    grid = (B // BM,)
    blk = lambda *shape: pl.BlockSpec(shape, lambda i: (i,) + (0,) * (len(shape) - 1))
    full = lambda a: pl.BlockSpec(a.shape, lambda i: (0,) * a.ndim)

    out = pl.pallas_call(
        _mpnn_body,
        grid=grid,
        in_specs=[
            blk(BM, N, 1), blk(BM, E, 1), blk(BM, E, 1), blk(BM, E, 1), blk(BM, E),
            full(aemb), full(bemb), full(memb), full(bdew),
            full(ew1), full(eb1), full(ew2), full(eb2),
            full(mw1), full(mb1), full(mw2), full(mb2),
            full(uw1), full(ub1), full(uw2), full(ub2),
        ],
        out_specs=blk(BM, NB, OUT_DIM),
        out_shape=jax.ShapeDtypeStruct((B, NB, OUT_DIM), jnp.float32),
        compiler_params=pltpu.CompilerParams(
            dimension_semantics=("arbitrary",),
        ),
    )(atom3, bond3, src3, dst3, bond_indices,
      aemb, bemb, memb, bdew,
      ew1, eb1, ew2, eb2, mw1, mb1, mw2, mb2, uw1, ub1, uw2, ub2)
    return out


# speedup vs baseline: 1.1172x; 1.1172x over previous
import jax
import jax.numpy as jnp
from jax.experimental import pallas as pl
from jax.experimental.pallas import tpu as pltpu

N = 64
E = 128
H = 128
NUM_MSG = 6
NUM_ATOM_TYPES = 171
NUM_BOND_TYPES = 200
OUT_DIM = 2
NB = E // 2
BM = 16

_BF = jnp.bfloat16
_DN = (((1,), (0,)), ((), ()))


def _dot(a, b):
    return jax.lax.dot_general(a.astype(_BF), b, _DN,
                               preferred_element_type=jnp.float32)


def _mpnn_body(atom3, bond3, src3, dst3, bidx,
               aemb_r, bemb_r, memb_r, bdew_r,
               ew1_r, eb1_r, ew2_r, eb2_r,
               mw1_r, mb1_r, mw2_r, mb2_r,
               uw1_r, ub1_r, uw2_r, ub2_r,
               out):
    f32 = jnp.float32
    relu = lambda x: jnp.maximum(x, 0.0)

    aemb = aemb_r[...]
    bemb = bemb_r[...]
    memb = memb_r[...]
    bdew = bdew_r[...]

    A3 = atom3[...]
    B3 = bond3[...]
    S3 = src3[...]
    D3 = dst3[...]
    BI = bidx[...]

    iota_an = jax.lax.broadcasted_iota(jnp.int32, (N, NUM_ATOM_TYPES), 1)
    iota_bn = jax.lax.broadcasted_iota(jnp.int32, (E, NUM_BOND_TYPES), 1)
    iota_en = jax.lax.broadcasted_iota(jnp.int32, (E, N), 1)
    iota_ne = jax.lax.broadcasted_iota(jnp.int32, (N, E), 0)
    iota_be = jax.lax.broadcasted_iota(jnp.int32, (NB, E), 0)

    atom_parts, bond_parts = [], []
    amask_parts, bmask_parts = [], []
    src_g, dst_g, src_s, agg_oh, mean_lk = [], [], [], [], []
    for m in range(BM):
        a = A3[m]
        b = B3[m]
        s = S3[m]
        d = D3[m]
        bi = BI[m:m + 1, :]
        a_oh = (iota_an == a).astype(_BF)
        b_oh = (iota_bn == b).astype(_BF)
        atom_parts.append(_dot(a_oh, aemb))
        bond_parts.append(_dot(b_oh, bemb))
        mean_lk.append(_dot(b_oh, memb))
        amask_parts.append((a != 0).astype(f32))
        bmask_parts.append((b != 0).astype(f32))
        src_g.append((iota_en == s).astype(_BF))
        dst_g.append((iota_en == d).astype(_BF))
        src_s.append((iota_ne == s.T).astype(_BF))
        agg_oh.append((iota_be == bi).astype(_BF))

    atom_state = jnp.concatenate(atom_parts, axis=0)
    bond_state = jnp.concatenate(bond_parts, axis=0)
    amask = jnp.concatenate(amask_parts, axis=0)
    bmask = jnp.concatenate(bmask_parts, axis=0)

    for i in range(NUM_MSG):
        W1 = ew1_r[i]
        src_atom = jnp.concatenate(
            [_dot(src_g[m], atom_state[m * N:(m + 1) * N].astype(_BF))
             for m in range(BM)], axis=0)
        dst_atom = jnp.concatenate(
            [_dot(dst_g[m], atom_state[m * N:(m + 1) * N].astype(_BF))
             for m in range(BM)], axis=0)
        h = relu(_dot(bond_state, W1[0:H])
                 + _dot(src_atom, W1[H:2 * H])
                 + _dot(dst_atom, W1[2 * H:3 * H])
                 + eb1_r[i])
        nb = _dot(h, ew2_r[i]) + eb2_r[i]
        bond_state = bond_state + nb * bmask
        if i < NUM_MSG - 1:
            M1 = mw1_r[i]
            h2 = relu(_dot(dst_atom, M1[0:H]) + _dot(bond_state, M1[H:2 * H])
                      + mb1_r[i])
            msg = (_dot(h2, mw2_r[i]) + mb2_r[i]) * bmask
            agg = jnp.concatenate(
                [_dot(src_s[m], msg[m * E:(m + 1) * E].astype(_BF))
                 for m in range(BM)], axis=0)
            na = relu(_dot(agg, uw1_r[i]) + ub1_r[i])
            na = _dot(na, uw2_r[i]) + ub2_r[i]
            atom_state = atom_state + na * amask

    masked = bond_state * bmask
    for m in range(BM):
        msl = masked[m * E:(m + 1) * E].astype(_BF)
        feat = _dot(agg_oh[m], msl)
        cnt = jnp.maximum(_dot(agg_oh[m], bmask_parts[m].astype(_BF)), 1.0)
        magg = _dot(agg_oh[m], (mean_lk[m] * bmask_parts[m]))
        out[m] = _dot(feat / cnt, bdew) + magg / cnt


@jax.jit
def kernel(atom, bond, connectivity, bond_indices, params):
    B = atom.shape[0]
    atom = atom.astype(jnp.int32)
    bond = bond.astype(jnp.int32)
    connectivity = connectivity.astype(jnp.int32)
    bond_indices = bond_indices.astype(jnp.int32)

    atom3 = atom.reshape(B, N, 1)
    bond3 = bond.reshape(B, E, 1)
    src3 = connectivity[:, :, 0].reshape(B, E, 1)
    dst3 = connectivity[:, :, 1].reshape(B, E, 1)

    bf = lambda x: x.astype(_BF)
    ew1 = bf(jnp.stack([p['w1'].T for p in params['edge']]))
    eb1 = jnp.stack([p['b1'].reshape(1, -1) for p in params['edge']])
    ew2 = bf(jnp.stack([p['w2'].T for p in params['edge']]))
    eb2 = jnp.stack([p['b2'].reshape(1, -1) for p in params['edge']])
    mw1 = bf(jnp.stack([p['mw1'].T for p in params['node']]))
    mb1 = jnp.stack([p['mb1'].reshape(1, -1) for p in params['node']])
    mw2 = bf(jnp.stack([p['mw2'].T for p in params['node']]))
    mb2 = jnp.stack([p['mb2'].reshape(1, -1) for p in params['node']])
    uw1 = bf(jnp.stack([p['uw1'].T for p in params['node']]))
    ub1 = jnp.stack([p['ub1'].reshape(1, -1) for p in params['node']])
    uw2 = bf(jnp.stack([p['uw2'].T for p in params['node']]))
    ub2 = jnp.stack([p['ub2'].reshape(1, -1) for p in params['node']])
    bdew = bf(params['bde_no_mean_w'].T)
    aemb = bf(params['atom_emb'])
    bemb = bf(params['bond_emb'])
    memb = bf(params['bde_mean_emb'])

    grid = (B // BM,)
    blk = lambda *shape: pl.BlockSpec(shape, lambda i: (i,) + (0,) * (len(shape) - 1))
    full = lambda a: pl.BlockSpec(a.shape, lambda i: (0,) * a.ndim)

    out = pl.pallas_call(
        _mpnn_body,
        grid=grid,
        in_specs=[
            blk(BM, N, 1), blk(BM, E, 1), blk(BM, E, 1), blk(BM, E, 1), blk(BM, E),
            full(aemb), full(bemb), full(memb), full(bdew),
            full(ew1), full(eb1), full(ew2), full(eb2),
            full(mw1), full(mb1), full(mw2), full(mb2),
            full(uw1), full(ub1), full(uw2), full(ub2),
        ],
        out_specs=blk(BM, NB, OUT_DIM),
        out_shape=jax.ShapeDtypeStruct((B, NB, OUT_DIM), jnp.float32),
        compiler_params=pltpu.CompilerParams(
            dimension_semantics=("arbitrary",),
        ),
    )(atom3, bond3, src3, dst3, bond_indices,
      aemb, bemb, memb, bdew,
      ew1, eb1, ew2, eb2, mw1, mb1, mw2, mb2, uw1, ub1, uw2, ub2)
    return out
